# baseline (speedup 1.0000x reference)
import functools
import math

import jax
import jax.numpy as jnp
from jax import lax
from jax.experimental import pallas as pl
from jax.experimental.pallas import tpu as pltpu
from jax.experimental.pallas import tpu_sc as plsc

_V = 100000
_N = 1024
_SMOOTH = 0.1
_CONF = 1.0 - _SMOOTH
_S = _SMOOTH / (_V - 2)
_W = _V - 100
_SLOGS = _S * math.log(_S)
_K0 = (_V - 2) * _SLOGS + _CONF * math.log(_CONF)

_RSC = 512
_NW = 32
_RPW = _RSC // _NW
_L = 16
_NACC = 8

_VA = 99968
_CC = 1408
_NCC = _VA // _CC
_BAND = 8
_NBAND = _RPW // _BAND

_WCC = _W // _CC
_WIN = _W - _WCC * _CC
_WBLK = (_WIN // _L) * _L
_WLANE = _WIN - _WBLK
assert _WCC == _NCC - 1

_BR = 8
_KB = 8
_RTC = _N - _RSC
_NC = _RTC // _BR


@functools.partial(
    pl.kernel,
    mesh=plsc.VectorSubcoreMesh(core_axis_name="c", subcore_axis_name="s"),
    out_type=jax.ShapeDtypeStruct((2 * _NW, _L), jnp.float32),
    scratch_types=[
        pltpu.VMEM((2, _BAND, _CC), jnp.float32),
        pltpu.VMEM((_L,), jnp.float32),
        pltpu.SemaphoreType.DMA,
        pltpu.SemaphoreType.DMA,
    ],
)
def _sc_stream_sum(x_hbm, out_hbm, buf, part_v, sem0, sem1):
    wid = lax.axis_index("s") * 2 + lax.axis_index("c")
    base = wid * _RPW
    lane = lax.iota(jnp.int32, _L)
    wmask = lane == _WLANE

    def start(band, cc, b, sem):
        row0 = pl.multiple_of(base + band * _BAND, 8)
        col0 = pl.multiple_of(cc * _CC, 128)
        pltpu.async_copy(
            x_hbm.at[pl.ds(row0, _BAND), pl.ds(col0, _CC)],
            buf.at[b],
            sem,
        ).start()

    def wait(b, sem):
        pltpu.async_copy(
            x_hbm.at[pl.ds(base, _BAND), pl.ds(0, _CC)], buf.at[b], sem
        ).wait()

    def accum(b, accs):
        def row_body(r, accs):
            accs = list(accs)
            for o in range(_CC // _L):
                accs[o % _NACC] = accs[o % _NACC] + buf[b, r, pl.ds(o * _L, _L)]
            return tuple(accs)

        return lax.fori_loop(0, _BAND, row_body, tuple(accs))

    def band_body(band, carry):
        accs, cw = carry[:_NACC], carry[_NACC]
        start(band, 0, 0, sem0)

        def pair_body(p, accs):
            start(band, 2 * p + 1, 1, sem1)
            wait(0, sem0)
            accs = accum(0, accs)
            start(band, 2 * p + 2, 0, sem0)
            wait(1, sem1)
            accs = accum(1, accs)
            return accs

        accs = lax.fori_loop(0, (_NCC - 1) // 2, pair_body, tuple(accs))
        wait(0, sem0)
        accs = accum(0, accs)

        def w_body(r, cw):
            return cw + jnp.where(wmask, buf[0, r, pl.ds(_WBLK, _L)], 0.0)

        cw = lax.fori_loop(0, _BAND, w_body, cw)
        return accs + (cw,)

    z = jnp.zeros((_L,), jnp.float32)
    carry = lax.fori_loop(0, _NBAND, band_body, (z,) * (_NACC + 1))
    acc = carry[0]
    for a in carry[1:_NACC]:
        acc = acc + a
    part_v[...] = acc
    pltpu.sync_copy(part_v, out_hbm.at[wid])
    part_v[...] = carry[_NACC]
    pltpu.sync_copy(part_v, out_hbm.at[_NW + wid])


def _tc_body(x_hbm, t2_ref, w2_ref, m_ref, ms_ref, ast_ref, rbk_ref,
             o_ref, buf, gbuf, sbuf, sems, gsem, ssem):
    strip_cp = pltpu.make_async_copy(
        x_hbm.at[pl.ds(0, _RSC), pl.ds(_VA, _V - _VA)], sbuf, ssem
    )
    strip_cp.start()

    def gloop(i, _):
        rb = pl.multiple_of(rbk_ref[i], 8)
        ast = pl.multiple_of(ast_ref[i], 128)
        pltpu.make_async_copy(
            x_hbm.at[pl.ds(rb, 8), pl.ds(ast, 128)],
            gbuf.at[pl.ds(i * 8, 8)],
            gsem,
        ).start()
        return 0

    lax.fori_loop(0, _RSC, gloop, 0)

    def copy(c, k):
        return pltpu.make_async_copy(
            x_hbm.at[pl.ds(_RSC + c * _BR, _BR)],
            buf.at[pl.ds(pl.multiple_of(k * _BR, 8), _BR)],
            sems.at[k],
        )

    for k in range(_KB):
        copy(k, k).start()

    def step(c, carry):
        tot, colw, ws = carry
        k = lax.rem(c, _KB)
        copy(c, k).wait()
        x = buf[pl.ds(pl.multiple_of(k * _BR, 8), _BR), :]
        tot += jnp.sum(x)
        colw += jnp.sum(x[:, _W])
        row0 = pl.multiple_of(_RSC + c * _BR, 8)
        t_blk = t2_ref[pl.ds(row0, _BR), :]
        w_blk = w2_ref[pl.ds(row0, _BR), :]
        col = lax.broadcasted_iota(jnp.int32, x.shape, 1)
        rowsel = jnp.sum(
            jnp.where(col == t_blk, x, 0.0), axis=1, keepdims=True
        )
        ws += jnp.sum(rowsel * w_blk)

        @pl.when(c + _KB < _NC)
        def _():
            copy(c + _KB, k).start()

        return tot, colw, ws

    z = jnp.float32(0.0)
    tot, colw, ws = lax.fori_loop(0, _NC, step, (z, z, z))

    pltpu.make_async_copy(
        x_hbm.at[pl.ds(0, _RSC * 8), pl.ds(0, 128)], gbuf, gsem
    ).wait()
    ws += jnp.sum(gbuf[...] * m_ref[...])

    strip_cp.wait()
    sb = sbuf[...]
    tot += jnp.sum(sb)
    ws += jnp.sum(sb * ms_ref[...])

    cnt = jnp.sum(jnp.where(t2_ref[...] == _W, 1.0, 0.0))
    o_ref[0, 0] = tot
    o_ref[0, 1] = colw
    o_ref[0, 2] = ws
    o_ref[0, 3] = cnt


def kernel(output, target):
    t = target
    t2 = t.reshape(_N, 1)
    w2 = (_S - _CONF) - _S * (t2 == _W).astype(jnp.float32)

    idx = jnp.arange(_RSC, dtype=jnp.int32)
    tsc = t[:_RSC]
    wsc = w2[:_RSC, 0]
    in_main = tsc < _VA
    astart = jnp.where(in_main, (tsc // 128) * 128, _VA - 128)
    rowblk = (idx // 8) * 8
    grow = 8 * idx + (idx % 8)
    gcol = tsc - astart
    m = jnp.zeros((_RSC * 8, 128), jnp.float32)
    m = m.at[grow, gcol].set(jnp.where(in_main, wsc, 0.0))
    scol = jnp.clip(tsc - _VA, 0, _V - _VA - 1)
    ms = jnp.zeros((_RSC, _V - _VA), jnp.float32)
    ms = ms.at[idx, scol].set(jnp.where(in_main, 0.0, wsc))

    parts = _sc_stream_sum(output)

    o = pl.pallas_call(
        _tc_body,
        in_specs=[
            pl.BlockSpec(memory_space=pl.ANY),
            pl.BlockSpec(memory_space=pltpu.VMEM),
            pl.BlockSpec(memory_space=pltpu.VMEM),
            pl.BlockSpec(memory_space=pltpu.VMEM),
            pl.BlockSpec(memory_space=pltpu.VMEM),
            pl.BlockSpec(memory_space=pltpu.SMEM),
            pl.BlockSpec(memory_space=pltpu.SMEM),
        ],
        out_specs=pl.BlockSpec(memory_space=pltpu.SMEM),
        out_shape=jax.ShapeDtypeStruct((1, 8), jnp.float32),
        scratch_shapes=[
            pltpu.VMEM((_KB * _BR, _V), jnp.float32),
            pltpu.VMEM((_RSC * 8, 128), jnp.float32),
            pltpu.VMEM((_RSC, _V - _VA), jnp.float32),
            pltpu.SemaphoreType.DMA((_KB,)),
            pltpu.SemaphoreType.DMA,
            pltpu.SemaphoreType.DMA,
        ],
    )(output, t2, w2, m, ms, astart, rowblk)

    tot = o[0, 0] + jnp.sum(parts[:_NW])
    colw = o[0, 1] + jnp.sum(parts[_NW:, _WLANE])
    return _N * _K0 + o[0, 3] * _SLOGS - _S * tot + _S * colw + o[0, 2]


# speedup vs baseline: 1.1538x; 1.1538x over previous
import functools
import math

import jax
import jax.numpy as jnp
from jax import lax
from jax.experimental import pallas as pl
from jax.experimental.pallas import tpu as pltpu
from jax.experimental.pallas import tpu_sc as plsc

_V = 100000
_N = 1024
_SMOOTH = 0.1
_CONF = 1.0 - _SMOOTH
_S = _SMOOTH / (_V - 2)
_W = _V - 100
_SLOGS = _S * math.log(_S)
_K0 = (_V - 2) * _SLOGS + _CONF * math.log(_CONF)

_RSC = 256
_NW = 32
_RPW = _RSC // _NW
_L = 16
_NACC = 8

_VA = 99968
_CC = 1408
_NCC = _VA // _CC
_BAND = 8
_NBAND = _RPW // _BAND

_WCC = _W // _CC
_WIN = _W - _WCC * _CC
_WBLK = (_WIN // _L) * _L
_WLANE = _WIN - _WBLK
assert _WCC == _NCC - 1

_BR = 8
_KB = 8
_RTC = _N - _RSC
_NC = _RTC // _BR


@functools.partial(
    pl.kernel,
    mesh=plsc.VectorSubcoreMesh(core_axis_name="c", subcore_axis_name="s"),
    out_type=jax.ShapeDtypeStruct((2 * _NW, _L), jnp.float32),
    scratch_types=[
        pltpu.VMEM((2, _BAND, _CC), jnp.float32),
        pltpu.VMEM((_L,), jnp.float32),
        pltpu.SemaphoreType.DMA,
        pltpu.SemaphoreType.DMA,
    ],
)
def _sc_stream_sum(x_hbm, out_hbm, buf, part_v, sem0, sem1):
    wid = lax.axis_index("s") * 2 + lax.axis_index("c")
    base = wid * _RPW
    lane = lax.iota(jnp.int32, _L)
    wmask = lane == _WLANE

    def start(band, cc, b, sem):
        row0 = pl.multiple_of(base + band * _BAND, 8)
        col0 = pl.multiple_of(cc * _CC, 128)
        pltpu.async_copy(
            x_hbm.at[pl.ds(row0, _BAND), pl.ds(col0, _CC)],
            buf.at[b],
            sem,
        ).start()

    def wait(b, sem):
        pltpu.async_copy(
            x_hbm.at[pl.ds(base, _BAND), pl.ds(0, _CC)], buf.at[b], sem
        ).wait()

    def accum(b, accs):
        def row_body(r, accs):
            accs = list(accs)
            for o in range(_CC // _L):
                accs[o % _NACC] = accs[o % _NACC] + buf[b, r, pl.ds(o * _L, _L)]
            return tuple(accs)

        return lax.fori_loop(0, _BAND, row_body, tuple(accs))

    def band_body(band, carry):
        accs, cw = carry[:_NACC], carry[_NACC]
        start(band, 0, 0, sem0)

        def pair_body(p, accs):
            start(band, 2 * p + 1, 1, sem1)
            wait(0, sem0)
            accs = accum(0, accs)
            start(band, 2 * p + 2, 0, sem0)
            wait(1, sem1)
            accs = accum(1, accs)
            return accs

        accs = lax.fori_loop(0, (_NCC - 1) // 2, pair_body, tuple(accs))
        wait(0, sem0)
        accs = accum(0, accs)

        def w_body(r, cw):
            return cw + jnp.where(wmask, buf[0, r, pl.ds(_WBLK, _L)], 0.0)

        cw = lax.fori_loop(0, _BAND, w_body, cw)
        return accs + (cw,)

    z = jnp.zeros((_L,), jnp.float32)
    carry = lax.fori_loop(0, _NBAND, band_body, (z,) * (_NACC + 1))
    acc = carry[0]
    for a in carry[1:_NACC]:
        acc = acc + a
    part_v[...] = acc
    pltpu.sync_copy(part_v, out_hbm.at[wid])
    part_v[...] = carry[_NACC]
    pltpu.sync_copy(part_v, out_hbm.at[_NW + wid])


def _tc_body(x_hbm, t2_ref, w2_ref, m_ref, ms_ref, ast_ref, rbk_ref,
             o_ref, buf, gbuf, sbuf, sems, gsem, ssem):
    strip_cp = pltpu.make_async_copy(
        x_hbm.at[pl.ds(0, _RSC), pl.ds(_VA, _V - _VA)], sbuf, ssem
    )
    strip_cp.start()

    def gloop(i, _):
        rb = pl.multiple_of(rbk_ref[i], 8)
        ast = pl.multiple_of(ast_ref[i], 128)
        pltpu.make_async_copy(
            x_hbm.at[pl.ds(rb, 8), pl.ds(ast, 128)],
            gbuf.at[pl.ds(i * 8, 8)],
            gsem,
        ).start()
        return 0

    lax.fori_loop(0, _RSC, gloop, 0)

    def copy(c, k):
        return pltpu.make_async_copy(
            x_hbm.at[pl.ds(_RSC + c * _BR, _BR)],
            buf.at[pl.ds(pl.multiple_of(k * _BR, 8), _BR)],
            sems.at[k],
        )

    for k in range(_KB):
        copy(k, k).start()

    def step(c, carry):
        tot, colw, ws = carry
        k = lax.rem(c, _KB)
        copy(c, k).wait()
        x = buf[pl.ds(pl.multiple_of(k * _BR, 8), _BR), :]
        tot += jnp.sum(x)
        colw += jnp.sum(x[:, _W])
        row0 = pl.multiple_of(_RSC + c * _BR, 8)
        t_blk = t2_ref[pl.ds(row0, _BR), :]
        w_blk = w2_ref[pl.ds(row0, _BR), :]
        col = lax.broadcasted_iota(jnp.int32, x.shape, 1)
        rowsel = jnp.sum(
            jnp.where(col == t_blk, x, 0.0), axis=1, keepdims=True
        )
        ws += jnp.sum(rowsel * w_blk)

        @pl.when(c + _KB < _NC)
        def _():
            copy(c + _KB, k).start()

        return tot, colw, ws

    z = jnp.float32(0.0)
    tot, colw, ws = lax.fori_loop(0, _NC, step, (z, z, z))

    pltpu.make_async_copy(
        x_hbm.at[pl.ds(0, _RSC * 8), pl.ds(0, 128)], gbuf, gsem
    ).wait()
    ws += jnp.sum(gbuf[...] * m_ref[...])

    strip_cp.wait()
    sb = sbuf[...]
    tot += jnp.sum(sb)
    ws += jnp.sum(sb * ms_ref[...])

    cnt = jnp.sum(jnp.where(t2_ref[...] == _W, 1.0, 0.0))
    o_ref[0, 0] = tot
    o_ref[0, 1] = colw
    o_ref[0, 2] = ws
    o_ref[0, 3] = cnt


def kernel(output, target):
    t = target
    t2 = t.reshape(_N, 1)
    w2 = (_S - _CONF) - _S * (t2 == _W).astype(jnp.float32)

    idx = jnp.arange(_RSC, dtype=jnp.int32)
    tsc = t[:_RSC]
    wsc = w2[:_RSC, 0]
    in_main = tsc < _VA
    astart = jnp.where(in_main, (tsc // 128) * 128, _VA - 128)
    rowblk = (idx // 8) * 8
    grow = 8 * idx + (idx % 8)
    gcol = tsc - astart
    m = jnp.zeros((_RSC * 8, 128), jnp.float32)
    m = m.at[grow, gcol].set(jnp.where(in_main, wsc, 0.0))
    scol = jnp.clip(tsc - _VA, 0, _V - _VA - 1)
    ms = jnp.zeros((_RSC, _V - _VA), jnp.float32)
    ms = ms.at[idx, scol].set(jnp.where(in_main, 0.0, wsc))

    o = pl.pallas_call(
        _tc_body,
        in_specs=[
            pl.BlockSpec(memory_space=pl.ANY),
            pl.BlockSpec(memory_space=pltpu.VMEM),
            pl.BlockSpec(memory_space=pltpu.VMEM),
            pl.BlockSpec(memory_space=pltpu.VMEM),
            pl.BlockSpec(memory_space=pltpu.VMEM),
            pl.BlockSpec(memory_space=pltpu.SMEM),
            pl.BlockSpec(memory_space=pltpu.SMEM),
        ],
        out_specs=pl.BlockSpec(memory_space=pltpu.SMEM),
        out_shape=jax.ShapeDtypeStruct((1, 8), jnp.float32),
        scratch_shapes=[
            pltpu.VMEM((_KB * _BR, _V), jnp.float32),
            pltpu.VMEM((_RSC * 8, 128), jnp.float32),
            pltpu.VMEM((_RSC, _V - _VA), jnp.float32),
            pltpu.SemaphoreType.DMA((_KB,)),
            pltpu.SemaphoreType.DMA,
            pltpu.SemaphoreType.DMA,
        ],
    )(output, t2, w2, m, ms, astart, rowblk)

    parts = _sc_stream_sum(output)

    tot = o[0, 0] + jnp.sum(parts[:_NW])
    colw = o[0, 1] + jnp.sum(parts[_NW:, _WLANE])
    return _N * _K0 + o[0, 3] * _SLOGS - _S * tot + _S * colw + o[0, 2]


# speedup vs baseline: 1.1614x; 1.0066x over previous
import functools
import math

import jax
import jax.numpy as jnp
from jax import lax
from jax.experimental import pallas as pl
from jax.experimental.pallas import tpu as pltpu
from jax.experimental.pallas import tpu_sc as plsc

_V = 100000
_N = 1024
_SMOOTH = 0.1
_CONF = 1.0 - _SMOOTH
_S = _SMOOTH / (_V - 2)
_W = _V - 100
_SLOGS = _S * math.log(_S)
_K0 = (_V - 2) * _SLOGS + _CONF * math.log(_CONF)

_RSC = 256
_NW = 32
_RPW = _RSC // _NW
_L = 16
_NACC = 8

_VA = 99968
_CC = 3072
_NCC = 32
_CR = _VA - _NCC * _CC
_BAND = 8
_NBAND = _RPW // _BAND

_WIN = _W - _NCC * _CC
_WBLK = (_WIN // _L) * _L
_WLANE = _WIN - _WBLK
assert 0 <= _WIN < _CR

_BR = 8
_KB = 8
_RTC = _N - _RSC
_NC = _RTC // _BR


@functools.partial(
    pl.kernel,
    mesh=plsc.VectorSubcoreMesh(core_axis_name="c", subcore_axis_name="s"),
    out_type=jax.ShapeDtypeStruct((2 * _NW, _L), jnp.float32),
    scratch_types=[
        pltpu.VMEM((2, _BAND, _CC), jnp.float32),
        pltpu.VMEM((_BAND, _CR), jnp.float32),
        pltpu.VMEM((_L,), jnp.float32),
        pltpu.SemaphoreType.DMA,
        pltpu.SemaphoreType.DMA,
        pltpu.SemaphoreType.DMA,
    ],
)
def _sc_stream_sum(x_hbm, out_hbm, buf, rbuf, part_v, sem0, sem1, sem2):
    wid = lax.axis_index("s") * 2 + lax.axis_index("c")
    base = wid * _RPW
    lane = lax.iota(jnp.int32, _L)
    wmask = lane == _WLANE

    def start(band, cc, b, sem):
        row0 = pl.multiple_of(base + band * _BAND, 8)
        col0 = pl.multiple_of(cc * _CC, 128)
        pltpu.async_copy(
            x_hbm.at[pl.ds(row0, _BAND), pl.ds(col0, _CC)],
            buf.at[b],
            sem,
        ).start()

    def wait(b, sem):
        pltpu.async_copy(
            x_hbm.at[pl.ds(base, _BAND), pl.ds(0, _CC)], buf.at[b], sem
        ).wait()

    def accum(load, nv, accs):
        def row_body(r, accs):
            accs = list(accs)
            for o in range(nv):
                accs[o % _NACC] = accs[o % _NACC] + load(r, o)
            return tuple(accs)

        return lax.fori_loop(0, _BAND, row_body, tuple(accs))

    def band_body(band, carry):
        accs, cw = carry[:_NACC], carry[_NACC]
        start(band, 0, 0, sem0)
        row0 = pl.multiple_of(base + band * _BAND, 8)
        rem_cp = pltpu.async_copy(
            x_hbm.at[pl.ds(row0, _BAND), pl.ds(_NCC * _CC, _CR)], rbuf, sem2
        )
        rem_cp.start()

        def load0(r, o):
            return buf[0, r, pl.ds(o * _L, _L)]

        def load1(r, o):
            return buf[1, r, pl.ds(o * _L, _L)]

        def loadr(r, o):
            return rbuf[r, pl.ds(o * _L, _L)]

        def pair_body(p, accs):
            start(band, 2 * p + 1, 1, sem1)
            wait(0, sem0)
            accs = accum(load0, _CC // _L, accs)

            @pl.when(2 * p + 2 < _NCC)
            def _():
                start(band, 2 * p + 2, 0, sem0)

            wait(1, sem1)
            accs = accum(load1, _CC // _L, accs)
            return accs

        accs = lax.fori_loop(0, _NCC // 2, pair_body, tuple(accs))
        rem_cp.wait()
        accs = accum(loadr, _CR // _L, accs)

        def w_body(r, cw):
            return cw + jnp.where(wmask, rbuf[r, pl.ds(_WBLK, _L)], 0.0)

        cw = lax.fori_loop(0, _BAND, w_body, cw)
        return accs + (cw,)

    z = jnp.zeros((_L,), jnp.float32)
    carry = lax.fori_loop(0, _NBAND, band_body, (z,) * (_NACC + 1))
    acc = carry[0]
    for a in carry[1:_NACC]:
        acc = acc + a
    part_v[...] = acc
    pltpu.sync_copy(part_v, out_hbm.at[wid])
    part_v[...] = carry[_NACC]
    pltpu.sync_copy(part_v, out_hbm.at[_NW + wid])


def _tc_body(x_hbm, t2_ref, w2_ref, m_ref, ms_ref, ast_ref, rbk_ref,
             o_ref, buf, gbuf, sbuf, sems, gsem, ssem):
    strip_cp = pltpu.make_async_copy(
        x_hbm.at[pl.ds(0, _RSC), pl.ds(_VA, _V - _VA)], sbuf, ssem
    )
    strip_cp.start()

    def gloop(i, _):
        rb = pl.multiple_of(rbk_ref[i], 8)
        ast = pl.multiple_of(ast_ref[i], 128)
        pltpu.make_async_copy(
            x_hbm.at[pl.ds(rb, 8), pl.ds(ast, 128)],
            gbuf.at[pl.ds(i * 8, 8)],
            gsem,
        ).start()
        return 0

    lax.fori_loop(0, _RSC, gloop, 0)

    def copy(c, k):
        return pltpu.make_async_copy(
            x_hbm.at[pl.ds(_RSC + c * _BR, _BR)],
            buf.at[pl.ds(pl.multiple_of(k * _BR, 8), _BR)],
            sems.at[k],
        )

    for k in range(_KB):
        copy(k, k).start()

    def step(c, carry):
        tot, colw, ws = carry
        k = lax.rem(c, _KB)
        copy(c, k).wait()
        x = buf[pl.ds(pl.multiple_of(k * _BR, 8), _BR), :]
        tot += jnp.sum(x)
        colw += jnp.sum(x[:, _W])
        row0 = pl.multiple_of(_RSC + c * _BR, 8)
        t_blk = t2_ref[pl.ds(row0, _BR), :]
        w_blk = w2_ref[pl.ds(row0, _BR), :]
        col = lax.broadcasted_iota(jnp.int32, x.shape, 1)
        rowsel = jnp.sum(
            jnp.where(col == t_blk, x, 0.0), axis=1, keepdims=True
        )
        ws += jnp.sum(rowsel * w_blk)

        @pl.when(c + _KB < _NC)
        def _():
            copy(c + _KB, k).start()

        return tot, colw, ws

    z = jnp.float32(0.0)
    tot, colw, ws = lax.fori_loop(0, _NC, step, (z, z, z))

    pltpu.make_async_copy(
        x_hbm.at[pl.ds(0, _RSC * 8), pl.ds(0, 128)], gbuf, gsem
    ).wait()
    ws += jnp.sum(gbuf[...] * m_ref[...])

    strip_cp.wait()
    sb = sbuf[...]
    tot += jnp.sum(sb)
    ws += jnp.sum(sb * ms_ref[...])

    cnt = jnp.sum(jnp.where(t2_ref[...] == _W, 1.0, 0.0))
    o_ref[0, 0] = tot
    o_ref[0, 1] = colw
    o_ref[0, 2] = ws
    o_ref[0, 3] = cnt


def kernel(output, target):
    t = target
    t2 = t.reshape(_N, 1)
    w2 = (_S - _CONF) - _S * (t2 == _W).astype(jnp.float32)

    idx = jnp.arange(_RSC, dtype=jnp.int32)
    tsc = t[:_RSC]
    wsc = w2[:_RSC, 0]
    in_main = tsc < _VA
    astart = jnp.where(in_main, (tsc // 128) * 128, _VA - 128)
    rowblk = (idx // 8) * 8
    grow = 8 * idx + (idx % 8)
    gcol = tsc - astart
    m = jnp.zeros((_RSC * 8, 128), jnp.float32)
    m = m.at[grow, gcol].set(jnp.where(in_main, wsc, 0.0))
    scol = jnp.clip(tsc - _VA, 0, _V - _VA - 1)
    ms = jnp.zeros((_RSC, _V - _VA), jnp.float32)
    ms = ms.at[idx, scol].set(jnp.where(in_main, 0.0, wsc))

    o = pl.pallas_call(
        _tc_body,
        in_specs=[
            pl.BlockSpec(memory_space=pl.ANY),
            pl.BlockSpec(memory_space=pltpu.VMEM),
            pl.BlockSpec(memory_space=pltpu.VMEM),
            pl.BlockSpec(memory_space=pltpu.VMEM),
            pl.BlockSpec(memory_space=pltpu.VMEM),
            pl.BlockSpec(memory_space=pltpu.SMEM),
            pl.BlockSpec(memory_space=pltpu.SMEM),
        ],
        out_specs=pl.BlockSpec(memory_space=pltpu.SMEM),
        out_shape=jax.ShapeDtypeStruct((1, 8), jnp.float32),
        scratch_shapes=[
            pltpu.VMEM((_KB * _BR, _V), jnp.float32),
            pltpu.VMEM((_RSC * 8, 128), jnp.float32),
            pltpu.VMEM((_RSC, _V - _VA), jnp.float32),
            pltpu.SemaphoreType.DMA((_KB,)),
            pltpu.SemaphoreType.DMA,
            pltpu.SemaphoreType.DMA,
        ],
    )(output, t2, w2, m, ms, astart, rowblk)

    parts = _sc_stream_sum(output)

    tot = o[0, 0] + jnp.sum(parts[:_NW])
    colw = o[0, 1] + jnp.sum(parts[_NW:, _WLANE])
    return _N * _K0 + o[0, 3] * _SLOGS - _S * tot + _S * colw + o[0, 2]
